# Initial kernel scaffold; baseline (speedup 1.0000x reference)
import jax
import jax.numpy as jnp
from jax.experimental import pallas as pl


def kernel(xs, edge_index, tm_w, tm_b, res_W0, res_b0, sage_Wl0, sage_Wr0, sage_b0, ssm_A0, ssm_B0, cm_W0, cm_b0, res_W1, res_b1, sage_Wl1, sage_Wr1, sage_b1, ssm_A1, ssm_B1, cm_W1, cm_b1, mlp_W, mlp_b):
    raise NotImplementedError("write your pallas kernel here")


# speedup vs baseline: 4.5468x; 4.5468x over previous
import functools

import jax
import jax.numpy as jnp
from jax import lax
from jax.experimental import pallas as pl
from jax.experimental.pallas import tpu as pltpu
from jax.experimental.pallas import tpu_sc as plsc

T, V, C, H, DS, E, OUT = 4, 10000, 128, 64, 16, 160000, 128
HDS = H * DS

NS = 16
K = 400
EP = E // NS
NCHUNK = EP // K

_MM = dict(preferred_element_type=jnp.float32,
           precision=jax.lax.Precision.HIGHEST)


@functools.cache
def _make_sc_agg():
  mesh = plsc.VectorSubcoreMesh(core_axis_name="c", subcore_axis_name="s")
  out_type = [
      jax.ShapeDtypeStruct((T, V, H), jnp.float32),
      jax.ShapeDtypeStruct((T, V, 16), jnp.float32),
  ]
  scratch = [
      pltpu.VMEM((K,), jnp.int32),
      pltpu.VMEM((K,), jnp.int32),
      pltpu.VMEM((K,), jnp.int32),
      pltpu.VMEM((K, H), jnp.float32),
      pltpu.VMEM((K, 16), jnp.float32),
      pltpu.SemaphoreType.DMA,
      pltpu.VMEM_SHARED((V, H), jnp.float32),
      pltpu.VMEM_SHARED((V, 16), jnp.float32),
  ]

  def body(feat_h, pk_h, zrows_h, zdeg_h, agg_h, deg_h,
           idx_pk, idx_s, idx_d, rows, ones_v, sem, agg_sh, deg_sh):
    c = lax.axis_index("c")
    s = lax.axis_index("s")

    def fill_ones(r, _):
      ones_v[r] = jnp.full((16,), 1.0, jnp.float32)
      return 0

    lax.fori_loop(0, K, fill_ones, 0)

    for tl in range(2):
      t = 2 * c + tl

      @pl.when(s < 10)
      def _():
        for k2 in range(8):
          pltpu.sync_copy(zrows_h,
                          agg_sh.at[pl.ds(s * 1000 + k2 * 125, 125)])
          pltpu.sync_copy(zdeg_h,
                          deg_sh.at[pl.ds(s * 1000 + k2 * 125, 125)])
      plsc.subcore_barrier()

      def chunk(i, _):
        base = t * E + s * EP + i * K
        pltpu.sync_copy(pk_h.at[pl.ds(base, K)], idx_pk)

        def unpack(j, _2):
          v = idx_pk[pl.ds(j * 16, 16)]
          idx_s[pl.ds(j * 16, 16)] = v & 0xFFFF
          idx_d[pl.ds(j * 16, 16)] = lax.shift_right_logical(v, 16)
          return 0

        lax.fori_loop(0, K // 16, unpack, 0)
        pltpu.async_copy(feat_h.at[idx_s], rows, sem).wait()
        pltpu.sync_copy(rows, agg_sh.at[idx_d], add=True)
        pltpu.sync_copy(ones_v, deg_sh.at[idx_d], add=True)
        return 0

      lax.fori_loop(0, NCHUNK, chunk, 0)
      plsc.subcore_barrier()

      @pl.when(s < 10)
      def _():
        pltpu.sync_copy(agg_sh.at[pl.ds(s * 1000, 1000)],
                        agg_h.at[t, pl.ds(s * 1000, 1000)])
        pltpu.sync_copy(deg_sh.at[pl.ds(s * 1000, 1000)],
                        deg_h.at[t, pl.ds(s * 1000, 1000)])
      plsc.subcore_barrier()

  return pl.kernel(
      body, out_type=out_type, mesh=mesh, scratch_types=scratch,
      compiler_params=pltpu.CompilerParams(use_tc_tiling_on_sc=False))


def _sc_aggregate(feat_flat, pk_idx):
  zrows = jnp.zeros((125, H), jnp.float32)
  zdeg = jnp.zeros((125, 16), jnp.float32)
  return _make_sc_agg()(feat_flat, pk_idx, zrows, zdeg)


def _p1_body(xs_ref, w_ref, b_ref, xm_ref):
  w = w_ref[...]
  b = b_ref[...]
  xs = [xs_ref[t] for t in range(T)]
  for t in range(T):
    xm = xs[t] * w[1] + b
    if t > 0:
      xm = xm + xs[t - 1] * w[0]
    if t < T - 1:
      xm = xm + xs[t + 1] * w[2]
    xm_ref[t] = xm


def _p1(xs, tm_w, tm_b):
  bV, grid = 2000, V // 2000
  w3 = jnp.transpose(tm_w[:, 0, :])
  return pl.pallas_call(
      _p1_body,
      grid=(grid,),
      in_specs=[
          pl.BlockSpec((T, bV, C), lambda i: (0, i, 0)),
          pl.BlockSpec((3, C), lambda i: (0, 0)),
          pl.BlockSpec((1, C), lambda i: (0, 0)),
      ],
      out_specs=pl.BlockSpec((T, bV, C), lambda i: (0, i, 0)),
      out_shape=jax.ShapeDtypeStruct((T, V, C), jnp.float32),
  )(xs, w3, tm_b[None, :])


def _pp_body(x_ref, wl_ref, xp_ref):
  wl = wl_ref[...]
  for t in range(T):
    xp_ref[t] = jnp.dot(x_ref[t], wl, **_MM)


def _pp(x, wl):
  bV, grid = 2000, V // 2000
  return pl.pallas_call(
      _pp_body,
      grid=(grid,),
      in_specs=[
          pl.BlockSpec((T, bV, C), lambda i: (0, i, 0)),
          pl.BlockSpec((C, H), lambda i: (0, 0)),
      ],
      out_specs=pl.BlockSpec((T, bV, H), lambda i: (0, i, 0)),
      out_shape=jax.ShapeDtypeStruct((T, V, H), jnp.float32),
  )(x, wl)


def _p3_body(xm_ref, agg_ref, deg_ref, wr_ref, rw_ref, sb_ref, rb_ref,
             av_ref, bv_ref, cw_ref, cb_ref, out_ref):
  wr, rw = wr_ref[...], rw_ref[...]
  sb, rb, cb = sb_ref[...], rb_ref[...], cb_ref[...]
  av, bv = av_ref[...], bv_ref[...]
  cw = cw_ref[...]
  bV = xm_ref.shape[1]
  st = jnp.zeros((bV, HDS), jnp.float32)
  zpad = jnp.zeros((bV, C - H), jnp.float32)
  for t in range(T):
    xm = xm_ref[t]
    scale = 1.0 / jnp.maximum(deg_ref[t][:, 0:1], 1.0)
    h = agg_ref[t] * scale + jnp.dot(xm, wr, **_MM) + sb
    xsr = jnp.dot(xm, rw, **_MM) + rb
    h16 = jnp.concatenate([h] * DS, axis=1)
    st = av * st + bv * h16
    y = jnp.dot(jnp.maximum(st, 0.0), cw, **_MM) + cb
    out_ref[t] = jnp.concatenate([y + xsr, zpad], axis=1)


def _p3(x, agg, deg, wr, rw, sb, rb, av, bv, cw, cb):
  bV, grid = 1000, V // 1000
  return pl.pallas_call(
      _p3_body,
      grid=(grid,),
      in_specs=[
          pl.BlockSpec((T, bV, C), lambda i: (0, i, 0)),
          pl.BlockSpec((T, bV, H), lambda i: (0, i, 0)),
          pl.BlockSpec((T, bV, 16), lambda i: (0, i, 0)),
          pl.BlockSpec((C, H), lambda i: (0, 0)),
          pl.BlockSpec((C, H), lambda i: (0, 0)),
          pl.BlockSpec((1, H), lambda i: (0, 0)),
          pl.BlockSpec((1, H), lambda i: (0, 0)),
          pl.BlockSpec((1, HDS), lambda i: (0, 0)),
          pl.BlockSpec((1, HDS), lambda i: (0, 0)),
          pl.BlockSpec((HDS, H), lambda i: (0, 0)),
          pl.BlockSpec((1, H), lambda i: (0, 0)),
      ],
      out_specs=pl.BlockSpec((T, bV, C), lambda i: (0, i, 0)),
      out_shape=jax.ShapeDtypeStruct((T, V, C), jnp.float32),
  )(x, agg, deg, wr, rw, sb, rb, av, bv, cw, cb)


def _p6_body(x_ref, mw_ref, mb_ref, out_ref):
  out_ref[...] = jnp.dot(x_ref[...][:, 0:H], mw_ref[...], **_MM) + mb_ref[...]


def _p6(x3, mlp_W, mlp_b):
  bV, grid = 2000, V // 2000
  return pl.pallas_call(
      _p6_body,
      grid=(grid,),
      in_specs=[
          pl.BlockSpec((bV, C), lambda i: (i, 0)),
          pl.BlockSpec((H, OUT), lambda i: (0, 0)),
          pl.BlockSpec((1, OUT), lambda i: (0, 0)),
      ],
      out_specs=pl.BlockSpec((bV, OUT), lambda i: (i, 0)),
      out_shape=jax.ShapeDtypeStruct((V, OUT), jnp.float32),
  )(x3, mlp_W, mlp_b[None, :])


def _pad_rows(w):
  return jnp.concatenate([w, jnp.zeros((C - H, H), jnp.float32)], axis=0)


def kernel(xs, edge_index, tm_w, tm_b, res_W0, res_b0, sage_Wl0, sage_Wr0,
           sage_b0, ssm_A0, ssm_B0, cm_W0, cm_b0, res_W1, res_b1, sage_Wl1,
           sage_Wr1, sage_b1, ssm_A1, ssm_B1, cm_W1, cm_b1, mlp_W, mlp_b):
  toff = (jnp.arange(T, dtype=jnp.int32) * V)[:, None]
  pk_idx = ((edge_index[:, 0, :] + toff)
            | (edge_index[:, 1, :] << 16)).reshape(T * E)

  cw0r = cm_W0.reshape(H, DS, H).transpose(1, 0, 2).reshape(HDS, H)
  cw1r = cm_W1.reshape(H, DS, H).transpose(1, 0, 2).reshape(HDS, H)
  wl_s = jnp.stack([sage_Wl0, _pad_rows(sage_Wl1)])
  wr_s = jnp.stack([sage_Wr0, _pad_rows(sage_Wr1)])
  rw_s = jnp.stack([res_W0, _pad_rows(res_W1)])
  sb_s = jnp.stack([sage_b0[None, :], sage_b1[None, :]])
  rb_s = jnp.stack([res_b0[None, :], res_b1[None, :]])
  av_s = jnp.stack([jnp.repeat(ssm_A0, H)[None, :],
                    jnp.repeat(ssm_A1, H)[None, :]])
  bv_s = jnp.stack([jnp.repeat(ssm_B0, H)[None, :],
                    jnp.repeat(ssm_B1, H)[None, :]])
  cw_s = jnp.stack([cw0r, cw1r])
  cb_s = jnp.stack([cm_b0[None, :], cm_b1[None, :]])

  xs_m = _p1(xs, tm_w, tm_b)

  wts_s = (wl_s, wr_s, rw_s, sb_s, rb_s, av_s, bv_s, cw_s, cb_s)

  n_layers = lax.optimization_barrier(jnp.int32(2))

  def cond(carry):
    return carry[0] < n_layers

  def layer(carry):
    i, x = carry
    wl, wr, rw, sb, rb, av, bv, cw, cb = (
        lax.dynamic_index_in_dim(w, i, 0, keepdims=False) for w in wts_s)
    xp = _pp(x, wl)
    agg, deg = _sc_aggregate(xp.reshape(T * V, H), pk_idx)
    out = _p3(x, agg, deg, wr, rw, sb, rb, av, bv, cw, cb)
    return (i + 1, out)

  _, x_fin = lax.while_loop(cond, layer, (jnp.int32(0), xs_m))

  return _p6(x_fin[T - 1], mlp_W, mlp_b)
